# initial kernel scaffold (unmeasured)
import jax
import jax.numpy as jnp
from jax import lax
from jax.experimental import pallas as pl
from jax.experimental.pallas import tpu as pltpu

N_DEV = 4
B, S, D = 1, 1024, 2048
DC = 128
H, DH, DR = 16, 128, 32
SQ = S // N_DEV
SCALE = (DH + DR) ** -0.5
F32 = jnp.float32


def kernel(x, Wdkv, Wuk, Wuv, Wq, Wqr, Wkr, Wo):
    def body(x_ref, wdkv_ref, wuk_ref, wuv_ref, wq_ref, wqr_ref, wkr_ref,
             wo_ref, out_ref,
             c_full, wuk_full, wuv_full,
             send_sems, recv_sems, out_send_sems, out_recv_sems):
        my = lax.axis_index("i")

        barrier_sem = pltpu.get_barrier_semaphore()
        for d in range(1, N_DEV):
            pl.semaphore_signal(
                barrier_sem, inc=1,
                device_id=((my + d) % N_DEV,),
                device_id_type=pl.DeviceIdType.MESH,
            )
        pl.semaphore_wait(barrier_sem, N_DEV - 1)

        x2 = x_ref[0]

        c_full[:, pl.ds(my * DC, DC)] = jnp.dot(
            x2, wdkv_ref[...], preferred_element_type=F32)
        wuk_full[pl.ds(my * DC, DC), :] = wuk_ref[...]
        wuv_full[pl.ds(my * DC, DC), :] = wuv_ref[...]

        sends = []
        for d in range(1, N_DEV):
            tgt = (my + d) % N_DEV
            slices = [
                c_full.at[:, pl.ds(my * DC, DC)],
                wuk_full.at[pl.ds(my * DC, DC), :],
                wuv_full.at[pl.ds(my * DC, DC), :],
            ]
            for a, sl in enumerate(slices):
                rdma = pltpu.make_async_remote_copy(
                    src_ref=sl,
                    dst_ref=sl,
                    send_sem=send_sems.at[d - 1, a],
                    recv_sem=recv_sems.at[d - 1, a],
                    device_id=(tgt,),
                    device_id_type=pl.DeviceIdType.MESH,
                )
                rdma.start()
                sends.append(rdma)

        q0 = my * SQ
        xq = lax.dynamic_slice_in_dim(x2, q0, SQ, axis=0)
        q = jnp.dot(xq, wq_ref[...], preferred_element_type=F32)
        qr = jnp.dot(xq, wqr_ref[...], preferred_element_type=F32)
        kr = jnp.dot(x2, wkr_ref[...], preferred_element_type=F32)

        for rdma in sends:
            rdma.wait_recv()
        for rdma in sends:
            rdma.wait_send()

        c = c_full[...]
        K = jnp.dot(c, wuk_full[...], preferred_element_type=F32)
        V = jnp.dot(c, wuv_full[...], preferred_element_type=F32)

        outs = []
        for h in range(H):
            qh = q[:, h * DH:(h + 1) * DH]
            kh = K[:, h * DH:(h + 1) * DH]
            vh = V[:, h * DH:(h + 1) * DH]
            qrh = qr[:, h * DR:(h + 1) * DR]
            s = lax.dot_general(qh, kh, (((1,), (1,)), ((), ())),
                                preferred_element_type=F32)
            s = s + lax.dot_general(qrh, kr, (((1,), (1,)), ((), ())),
                                    preferred_element_type=F32)
            s = s * SCALE
            m = jnp.max(s, axis=1, keepdims=True)
            p = jnp.exp(s - m)
            p = p / jnp.sum(p, axis=1, keepdims=True)
            outs.append(jnp.dot(p, vh, preferred_element_type=F32))
        o = jnp.concatenate(outs, axis=1)
        out_q = jnp.dot(o, wo_ref[...], preferred_element_type=F32)

        out_ref[0, pl.ds(q0, SQ), :] = out_q

        out_sends = []
        for d in range(1, N_DEV):
            tgt = (my + d) % N_DEV
            sl = out_ref.at[:, pl.ds(q0, SQ), :]
            rdma = pltpu.make_async_remote_copy(
                src_ref=sl,
                dst_ref=sl,
                send_sem=out_send_sems.at[d - 1],
                recv_sem=out_recv_sems.at[d - 1],
                device_id=(tgt,),
                device_id_type=pl.DeviceIdType.MESH,
            )
            rdma.start()
            out_sends.append(rdma)

        for rdma in out_sends:
            rdma.wait_recv()
        for rdma in out_sends:
            rdma.wait_send()

    return pl.pallas_call(
        body,
        out_shape=jax.ShapeDtypeStruct((B, S, D), F32),
        in_specs=[pl.BlockSpec(memory_space=pltpu.VMEM)] * 8,
        out_specs=pl.BlockSpec(memory_space=pltpu.VMEM),
        scratch_shapes=[
            pltpu.VMEM((S, N_DEV * DC), F32),
            pltpu.VMEM((N_DEV * DC, D), F32),
            pltpu.VMEM((N_DEV * DC, D), F32),
            pltpu.SemaphoreType.DMA((N_DEV - 1, 3)),
            pltpu.SemaphoreType.DMA((N_DEV - 1, 3)),
            pltpu.SemaphoreType.DMA((N_DEV - 1,)),
            pltpu.SemaphoreType.DMA((N_DEV - 1,)),
        ],
        compiler_params=pltpu.CompilerParams(collective_id=0),
    )(x, Wdkv, Wuk, Wuv, Wq, Wqr, Wkr, Wo)


# baseline (device time: 162666 ns/iter reference)
import jax
import jax.numpy as jnp
from jax import lax
from jax.experimental import pallas as pl
from jax.experimental.pallas import tpu as pltpu

N_DEV = 4
B, S, D = 1, 1024, 2048
DC = 128
H, DH, DR = 16, 128, 32
SQ = S // N_DEV
SB = 256
SCALE = (DH + DR) ** -0.5
F32 = jnp.float32

_VMEM = pl.BlockSpec(memory_space=pltpu.VMEM)


def _signal_peers(sem, my):
    for d in range(1, N_DEV):
        pl.semaphore_signal(
            sem, inc=1,
            device_id=((my + d) % N_DEV,),
            device_id_type=pl.DeviceIdType.MESH,
        )


def _gather_kv(x, Wdkv, Wuk, Wuv, Wkr):

    def body(x_ref, wdkv_ref, wuk_ref, wuv_ref, wkr_ref,
             k_ref, v_ref, kr_ref,
             c_full, wuk_full, wuv_full, send_sems, recv_sems):
        my = lax.axis_index("i")
        barrier_sem = pltpu.get_barrier_semaphore()
        _signal_peers(barrier_sem, my)

        for b in range(S // SB):
            xb = x_ref[0, b * SB:(b + 1) * SB, :]
            c_full[b * SB:(b + 1) * SB, pl.ds(my * DC, DC)] = jnp.dot(
                xb, wdkv_ref[...], preferred_element_type=F32)
            kr_ref[b * SB:(b + 1) * SB, :] = jnp.dot(
                xb, wkr_ref[...], preferred_element_type=F32)
        wuk_full[pl.ds(my * DC, DC), :] = wuk_ref[...]
        wuv_full[pl.ds(my * DC, DC), :] = wuv_ref[...]

        pl.semaphore_wait(barrier_sem, N_DEV - 1)

        sends = []
        for d in range(1, N_DEV):
            tgt = (my + d) % N_DEV
            slices = [
                c_full.at[:, pl.ds(my * DC, DC)],
                wuk_full.at[pl.ds(my * DC, DC), :],
                wuv_full.at[pl.ds(my * DC, DC), :],
            ]
            for a, sl in enumerate(slices):
                rdma = pltpu.make_async_remote_copy(
                    src_ref=sl,
                    dst_ref=sl,
                    send_sem=send_sems.at[d - 1, a],
                    recv_sem=recv_sems.at[d - 1, a],
                    device_id=(tgt,),
                    device_id_type=pl.DeviceIdType.MESH,
                )
                rdma.start()
                sends.append(rdma)

        for rdma in sends:
            rdma.wait_recv()
        for rdma in sends:
            rdma.wait_send()

        for b in range(S // SB):
            cb = c_full[b * SB:(b + 1) * SB, :]
            k_ref[b * SB:(b + 1) * SB, :] = jnp.dot(
                cb, wuk_full[...], preferred_element_type=F32)
            v_ref[b * SB:(b + 1) * SB, :] = jnp.dot(
                cb, wuv_full[...], preferred_element_type=F32)

    return pl.pallas_call(
        body,
        out_shape=[
            jax.ShapeDtypeStruct((S, H * DH), F32),
            jax.ShapeDtypeStruct((S, H * DH), F32),
            jax.ShapeDtypeStruct((S, DR), F32),
        ],
        in_specs=[_VMEM] * 5,
        out_specs=[_VMEM] * 3,
        scratch_shapes=[
            pltpu.VMEM((S, N_DEV * DC), F32),
            pltpu.VMEM((N_DEV * DC, D), F32),
            pltpu.VMEM((N_DEV * DC, D), F32),
            pltpu.SemaphoreType.DMA((N_DEV - 1, 3)),
            pltpu.SemaphoreType.DMA((N_DEV - 1, 3)),
        ],
        compiler_params=pltpu.CompilerParams(collective_id=0),
    )(x, Wdkv, Wuk, Wuv, Wkr)


def _attention(xq, Wq, Wqr, kr, K, V):

    def body(xq_ref, wq_ref, wqr_ref, kr_ref, k_ref, v_ref, o_ref):
        q = jnp.dot(xq_ref[...], wq_ref[...], preferred_element_type=F32)
        qr = jnp.dot(xq_ref[...], wqr_ref[...], preferred_element_type=F32)
        kr = kr_ref[...]
        for h in range(H):
            qh = q[:, h * DH:(h + 1) * DH]
            kh = k_ref[:, h * DH:(h + 1) * DH]
            vh = v_ref[:, h * DH:(h + 1) * DH]
            qrh = qr[:, h * DR:(h + 1) * DR]
            s = lax.dot_general(qh, kh, (((1,), (1,)), ((), ())),
                                preferred_element_type=F32)
            s = s + lax.dot_general(qrh, kr, (((1,), (1,)), ((), ())),
                                    preferred_element_type=F32)
            s = s * SCALE
            m = jnp.max(s, axis=1, keepdims=True)
            p = jnp.exp(s - m)
            p = p / jnp.sum(p, axis=1, keepdims=True)
            o_ref[:, h * DH:(h + 1) * DH] = jnp.dot(
                p, vh, preferred_element_type=F32)

    return pl.pallas_call(
        body,
        out_shape=jax.ShapeDtypeStruct((SQ, H * DH), F32),
        in_specs=[_VMEM] * 6,
        out_specs=_VMEM,
    )(xq, Wq, Wqr, kr, K, V)


def _out_proj_bcast(o, Wo):

    def body(o_ref, wo_ref, out_ref, send_sems, recv_sems):
        my = lax.axis_index("i")
        q0 = my * SQ
        barrier_sem = pltpu.get_barrier_semaphore()
        _signal_peers(barrier_sem, my)

        out_ref[0, pl.ds(q0, SQ), :] = jnp.dot(
            o_ref[...], wo_ref[...], preferred_element_type=F32)

        pl.semaphore_wait(barrier_sem, N_DEV - 1)
        sends = []
        for d in range(1, N_DEV):
            tgt = (my + d) % N_DEV
            sl = out_ref.at[:, pl.ds(q0, SQ), :]
            rdma = pltpu.make_async_remote_copy(
                src_ref=sl,
                dst_ref=sl,
                send_sem=send_sems.at[d - 1],
                recv_sem=recv_sems.at[d - 1],
                device_id=(tgt,),
                device_id_type=pl.DeviceIdType.MESH,
            )
            rdma.start()
            sends.append(rdma)

        for rdma in sends:
            rdma.wait_recv()
        for rdma in sends:
            rdma.wait_send()

    return pl.pallas_call(
        body,
        out_shape=jax.ShapeDtypeStruct((B, S, D), F32),
        in_specs=[_VMEM] * 2,
        out_specs=_VMEM,
        scratch_shapes=[
            pltpu.SemaphoreType.DMA((N_DEV - 1,)),
            pltpu.SemaphoreType.DMA((N_DEV - 1,)),
        ],
        compiler_params=pltpu.CompilerParams(collective_id=1),
    )(o, Wo)


def kernel(x, Wdkv, Wuk, Wuv, Wq, Wqr, Wkr, Wo):
    my = lax.axis_index("i")
    K, V, kr = _gather_kv(x, Wdkv, Wuk, Wuv, Wkr)
    xq = lax.dynamic_slice(x, (0, my * SQ, 0), (1, SQ, D))[0]
    o = _attention(xq, Wq, Wqr, kr, K, V)
    return _out_proj_bcast(o, Wo)


# device time: 108674 ns/iter; 1.4968x vs baseline; 1.4968x over previous
import jax
import jax.numpy as jnp
from jax import lax
from jax.experimental import pallas as pl
from jax.experimental.pallas import tpu as pltpu

N_DEV = 4
B, S, D = 1, 1024, 2048
DC = 128
H, DH, DR = 16, 128, 32
SQ = S // N_DEV
SB = 256
SCALE = (DH + DR) ** -0.5
F32 = jnp.float32
BF = jnp.bfloat16

_VMEM = pl.BlockSpec(memory_space=pltpu.VMEM)


def _signal_peers(sem, my):
    for d in range(1, N_DEV):
        pl.semaphore_signal(
            sem, inc=1,
            device_id=((my + d) % N_DEV,),
            device_id_type=pl.DeviceIdType.MESH,
        )


def _gather_kv(x, Wdkv, Wuk, Wuv, Wkr):

    def body(x_ref, wdkv_ref, wuk_ref, wuv_ref, wkr_ref,
             k_ref, v_ref, kr_ref,
             c_full, wuk_full, wuv_full, send_sems, recv_sems):
        my = lax.axis_index("i")
        barrier_sem = pltpu.get_barrier_semaphore()
        _signal_peers(barrier_sem, my)

        wdkv_b = wdkv_ref[...].astype(BF)
        wkr_b = wkr_ref[...].astype(BF)
        for b in range(S // SB):
            xb = x_ref[0, b * SB:(b + 1) * SB, :].astype(BF)
            c_full[b * SB:(b + 1) * SB, pl.ds(my * DC, DC)] = jnp.dot(
                xb, wdkv_b, preferred_element_type=F32).astype(BF)
            kr_ref[b * SB:(b + 1) * SB, :] = jnp.dot(
                xb, wkr_b, preferred_element_type=F32).astype(BF)
        wuk_full[pl.ds(my * DC, DC), :] = wuk_ref[...].astype(BF)
        wuv_full[pl.ds(my * DC, DC), :] = wuv_ref[...].astype(BF)

        pl.semaphore_wait(barrier_sem, N_DEV - 1)

        sends = []
        for d in range(1, N_DEV):
            tgt = (my + d) % N_DEV
            slices = [
                c_full.at[:, pl.ds(my * DC, DC)],
                wuk_full.at[pl.ds(my * DC, DC), :],
                wuv_full.at[pl.ds(my * DC, DC), :],
            ]
            for a, sl in enumerate(slices):
                rdma = pltpu.make_async_remote_copy(
                    src_ref=sl,
                    dst_ref=sl,
                    send_sem=send_sems.at[d - 1, a],
                    recv_sem=recv_sems.at[d - 1, a],
                    device_id=(tgt,),
                    device_id_type=pl.DeviceIdType.MESH,
                )
                rdma.start()
                sends.append(rdma)

        for rdma in sends:
            rdma.wait_recv()
        for rdma in sends:
            rdma.wait_send()

        for b in range(S // SB):
            cb = c_full[b * SB:(b + 1) * SB, :]
            k_ref[b * SB:(b + 1) * SB, :] = jnp.dot(
                cb, wuk_full[...], preferred_element_type=F32).astype(BF)
            v_ref[b * SB:(b + 1) * SB, :] = jnp.dot(
                cb, wuv_full[...], preferred_element_type=F32).astype(BF)

    return pl.pallas_call(
        body,
        out_shape=[
            jax.ShapeDtypeStruct((S, H * DH), BF),
            jax.ShapeDtypeStruct((S, H * DH), BF),
            jax.ShapeDtypeStruct((S, DR), BF),
        ],
        in_specs=[_VMEM] * 5,
        out_specs=[_VMEM] * 3,
        scratch_shapes=[
            pltpu.VMEM((S, N_DEV * DC), BF),
            pltpu.VMEM((N_DEV * DC, D), BF),
            pltpu.VMEM((N_DEV * DC, D), BF),
            pltpu.SemaphoreType.DMA((N_DEV - 1, 3)),
            pltpu.SemaphoreType.DMA((N_DEV - 1, 3)),
        ],
        compiler_params=pltpu.CompilerParams(collective_id=0),
    )(x, Wdkv, Wuk, Wuv, Wkr)


def _attention(xq, Wq, Wqr, kr, K, V):

    def body(xq_ref, wq_ref, wqr_ref, kr_ref, k_ref, v_ref, o_ref):
        xq = xq_ref[...].astype(BF)
        q = jnp.dot(xq, wq_ref[...].astype(BF),
                    preferred_element_type=F32).astype(BF)
        qr = jnp.dot(xq, wqr_ref[...].astype(BF),
                     preferred_element_type=F32).astype(BF)
        kr = kr_ref[...]
        for h in range(H):
            qh = q[:, h * DH:(h + 1) * DH]
            kh = k_ref[:, h * DH:(h + 1) * DH]
            vh = v_ref[:, h * DH:(h + 1) * DH]
            qrh = qr[:, h * DR:(h + 1) * DR]
            s = lax.dot_general(qh, kh, (((1,), (1,)), ((), ())),
                                preferred_element_type=F32)
            s = s + lax.dot_general(qrh, kr, (((1,), (1,)), ((), ())),
                                    preferred_element_type=F32)
            s = s * SCALE
            m = jnp.max(s, axis=1, keepdims=True)
            p = jnp.exp(s - m)
            p = (p / jnp.sum(p, axis=1, keepdims=True)).astype(BF)
            o_ref[:, h * DH:(h + 1) * DH] = jnp.dot(
                p, vh, preferred_element_type=F32).astype(BF)

    return pl.pallas_call(
        body,
        out_shape=jax.ShapeDtypeStruct((SQ, H * DH), BF),
        in_specs=[_VMEM] * 6,
        out_specs=_VMEM,
    )(xq, Wq, Wqr, kr, K, V)


def _out_proj_bcast(o, Wo):

    def body(o_ref, wo_ref, out_ref, out_bf, send_sems, recv_sems):
        my = lax.axis_index("i")
        q0 = my * SQ
        barrier_sem = pltpu.get_barrier_semaphore()
        _signal_peers(barrier_sem, my)

        out_q = jnp.dot(o_ref[...], wo_ref[...].astype(BF),
                        preferred_element_type=F32)
        out_ref[0, pl.ds(q0, SQ), :] = out_q
        out_bf[pl.ds(q0, SQ), :] = out_q.astype(BF)

        pl.semaphore_wait(barrier_sem, N_DEV - 1)
        sends = []
        for d in range(1, N_DEV):
            tgt = (my + d) % N_DEV
            sl = out_bf.at[pl.ds(q0, SQ), :]
            rdma = pltpu.make_async_remote_copy(
                src_ref=sl,
                dst_ref=sl,
                send_sem=send_sems.at[d - 1],
                recv_sem=recv_sems.at[d - 1],
                device_id=(tgt,),
                device_id_type=pl.DeviceIdType.MESH,
            )
            rdma.start()
            sends.append(rdma)

        for rdma in sends:
            rdma.wait_recv()
        for d in range(1, N_DEV):
            p0 = ((my + N_DEV - d) % N_DEV) * SQ
            out_ref[0, pl.ds(p0, SQ), :] = out_bf[pl.ds(p0, SQ), :].astype(F32)
        for rdma in sends:
            rdma.wait_send()

    return pl.pallas_call(
        body,
        out_shape=jax.ShapeDtypeStruct((B, S, D), F32),
        in_specs=[_VMEM] * 2,
        out_specs=_VMEM,
        scratch_shapes=[
            pltpu.VMEM((S, D), BF),
            pltpu.SemaphoreType.DMA((N_DEV - 1,)),
            pltpu.SemaphoreType.DMA((N_DEV - 1,)),
        ],
        compiler_params=pltpu.CompilerParams(collective_id=1),
    )(o, Wo)


def kernel(x, Wdkv, Wuk, Wuv, Wq, Wqr, Wkr, Wo):
    my = lax.axis_index("i")
    K, V, kr = _gather_kv(x, Wdkv, Wuk, Wuv, Wkr)
    xq = lax.dynamic_slice(x, (0, my * SQ, 0), (1, SQ, D))[0]
    o = _attention(xq, Wq, Wqr, kr, K, V)
    return _out_proj_bcast(o, Wo)
